# Initial kernel scaffold; baseline (speedup 1.0000x reference)
import jax
import jax.numpy as jnp
from jax.experimental import pallas as pl


def kernel(h, pos, edge_attr, cond, msg_w1, msg_b1, msg_w2, msg_b2, coord_w1, coord_b1, coord_w2, node_w1, node_b1, node_w2, node_b2, film_w, film_b, ln_g, ln_b, edge_index, batch):
    raise NotImplementedError("write your pallas kernel here")


# speedup vs baseline: 4.7730x; 4.7730x over previous
import functools

import jax
import jax.numpy as jnp
from jax import lax
from jax.experimental import pallas as pl
from jax.experimental.pallas import tpu as pltpu
from jax.experimental.pallas import tpu_sc as plsc

N = 10000
E = 320000
H = 128
ED = 16
G = 64
CE = 128
NW = 32
FULL = (E // NW) // CE
EPW = FULL * CE
TAIL_BASE = NW * EPW
TAIL_CHUNKS = (E - TAIL_BASE) // CE
NG = CE // 16

BE = 2560
BN = 1000
NPAD = 10240
RPT = NPAD // 16
ZR = 128

_f32 = jnp.float32


def _silu(x):
    return x * jax.lax.logistic(x)


def _tables_body(h_ref, w1a_ref, w1b_ref, b1_ref, ta_ref, tb_ref):
    hb = h_ref[...]
    ta_ref[...] = jnp.dot(hb, w1a_ref[...], preferred_element_type=_f32) + b1_ref[...]
    tb_ref[...] = jnp.dot(hb, w1b_ref[...], preferred_element_type=_f32)


def _make_tables(h, w1a, w1b, b1):
    return pl.pallas_call(
        _tables_body,
        grid=(N // BN,),
        in_specs=[
            pl.BlockSpec((BN, H), lambda i: (i, 0)),
            pl.BlockSpec((H, H), lambda i: (0, 0)),
            pl.BlockSpec((H, H), lambda i: (0, 0)),
            pl.BlockSpec((1, H), lambda i: (0, 0)),
        ],
        out_specs=[
            pl.BlockSpec((BN, H), lambda i: (i, 0)),
            pl.BlockSpec((BN, H), lambda i: (i, 0)),
        ],
        out_shape=[
            jax.ShapeDtypeStruct((N, H), _f32),
            jax.ShapeDtypeStruct((N, H), _f32),
        ],
    )(h, w1a, w1b, b1)


def _gather_body(ta, tb, pxh, pyh, pzh, ii, jj, ga, gb, rel8,
                 idxi, idxj, bufa, bufb, bufr,
                 pxi, pyi, pzi, pxj, pyj, pzj, sema, semb, semp):
    c = lax.axis_index("c")
    s = lax.axis_index("s")
    wid = s * 2 + c

    for d in range(4, 8):
        for g in range(NG):
            bufr[d, pl.ds(g * 16, 16)] = jnp.zeros((16,), _f32)

    def do_chunk(base):
        base = pl.multiple_of(base, CE)
        pltpu.sync_copy(ii.at[pl.ds(base, CE)], idxi)
        pltpu.sync_copy(jj.at[pl.ds(base, CE)], idxj)
        cpa = pltpu.async_copy(ta.at[idxi], bufa, sema)
        cpb = pltpu.async_copy(tb.at[idxj], bufb, semb)
        cps = [pltpu.async_copy(src.at[idx], dst, semp)
               for src, idx, dst in ((pxh, idxi, pxi), (pyh, idxi, pyi),
                                     (pzh, idxi, pzi), (pxh, idxj, pxj),
                                     (pyh, idxj, pyj), (pzh, idxj, pzj))]
        for cp in cps:
            cp.wait()
        for g in range(NG):
            d16 = pl.ds(g * 16, 16)
            rx = pxi[d16] - pxj[d16]
            ry = pyi[d16] - pyj[d16]
            rz = pzi[d16] - pzj[d16]
            bufr[0, d16] = rx
            bufr[1, d16] = ry
            bufr[2, d16] = rz
            bufr[3, d16] = rx * rx + ry * ry + rz * rz
        pltpu.sync_copy(bufr, rel8.at[:, pl.ds(base, CE)])
        cpa.wait()
        cpb.wait()
        pltpu.sync_copy(bufa, ga.at[pl.ds(base, CE)])
        pltpu.sync_copy(bufb, gb.at[pl.ds(base, CE)])

    def loop_body(t, carry):
        do_chunk(wid * EPW + t * CE)
        return carry

    lax.fori_loop(0, FULL, loop_body, 0)

    @pl.when(wid < TAIL_CHUNKS)
    def _():
        do_chunk(TAIL_BASE + wid * CE)


def _sc_gather(ta, tb, pxh, pyh, pzh, ii, jj):
    mesh = plsc.VectorSubcoreMesh(core_axis_name="c", subcore_axis_name="s")
    f = pl.kernel(
        _gather_body,
        mesh=mesh,
        out_type=[
            jax.ShapeDtypeStruct((E, H), _f32),
            jax.ShapeDtypeStruct((E, H), _f32),
            jax.ShapeDtypeStruct((8, E), _f32),
        ],
        scratch_types=[
            pltpu.VMEM((CE,), jnp.int32),
            pltpu.VMEM((CE,), jnp.int32),
            pltpu.VMEM((CE, H), _f32),
            pltpu.VMEM((CE, H), _f32),
            pltpu.VMEM((8, CE), _f32),
            pltpu.VMEM((CE,), _f32),
            pltpu.VMEM((CE,), _f32),
            pltpu.VMEM((CE,), _f32),
            pltpu.VMEM((CE,), _f32),
            pltpu.VMEM((CE,), _f32),
            pltpu.VMEM((CE,), _f32),
            pltpu.SemaphoreType.DMA,
            pltpu.SemaphoreType.DMA,
            pltpu.SemaphoreType.DMA,
        ],
    )
    return f(ta, tb, pxh, pyh, pzh, ii, jj)


def _edge_body(ga_ref, gb_ref, r8_ref, ea_ref, wd8_ref, w1e_ref, w2_ref,
               b2_ref, cw1_ref, cb1_ref, cw2_ref, m_ref, cu_ref):
    r8 = r8_ref[...]
    pre = (ga_ref[...] + gb_ref[...]
           + jnp.dot(ea_ref[...], w1e_ref[...], preferred_element_type=_f32)
           + lax.dot_general(r8, wd8_ref[...], (((0,), (0,)), ((), ())),
                             preferred_element_type=_f32))
    m1 = _silu(pre)
    m2 = _silu(jnp.dot(m1, w2_ref[...], preferred_element_type=_f32) + b2_ref[...])
    t = _silu(jnp.dot(m2, cw1_ref[...], preferred_element_type=_f32) + cb1_ref[...])
    cw_row = lax.dot_general(cw2_ref[...], t, (((1,), (1,)), ((), ())),
                             preferred_element_type=_f32)
    dsq_row = r8[3:4, :]
    scale = cw_row * lax.rsqrt(dsq_row + 1e-8)
    cu = r8[0:3, :] * scale
    m_ref[...] = m2
    cu_ref[...] = jnp.concatenate(
        [cu, jnp.zeros((5, cu.shape[1]), _f32)], axis=0)


def _edge_mlp(ga, gb, rel8, ea, wd8, w1e, w2, b2, cw1, cb1, cw2):
    full = lambda i: (0, 0)
    return pl.pallas_call(
        _edge_body,
        grid=(E // BE,),
        in_specs=[
            pl.BlockSpec((BE, H), lambda i: (i, 0)),
            pl.BlockSpec((BE, H), lambda i: (i, 0)),
            pl.BlockSpec((8, BE), lambda i: (0, i)),
            pl.BlockSpec((BE, ED), lambda i: (i, 0)),
            pl.BlockSpec((8, H), full),
            pl.BlockSpec((ED, H), full),
            pl.BlockSpec((H, H), full),
            pl.BlockSpec((1, H), full),
            pl.BlockSpec((H, H), full),
            pl.BlockSpec((1, H), full),
            pl.BlockSpec((1, H), full),
        ],
        out_specs=[
            pl.BlockSpec((BE, H), lambda i: (i, 0)),
            pl.BlockSpec((8, BE), lambda i: (0, i)),
        ],
        out_shape=[
            jax.ShapeDtypeStruct((E, H), _f32),
            jax.ShapeDtypeStruct((8, E), _f32),
        ],
    )(ga, gb, rel8, ea, wd8, w1e, w2, b2, cw1, cb1, cw2)


def _scatter_body(m, cu8, ii, parts, cpx, cpy, cpz,
                  idxv, mbuf, cbuf, zbuf, zbuf1, acc, accx, accy, accz):
    c = lax.axis_index("c")
    s = lax.axis_index("s")
    wid = s * 2 + c

    def zb_body(r, carry):
        for k in range(H // 16):
            zbuf[r, pl.ds(k * 16, 16)] = jnp.zeros((16,), _f32)
        return carry

    lax.fori_loop(0, ZR, zb_body, 0)

    def z1_body(r, carry):
        zbuf1[pl.ds(r * 16, 16)] = jnp.zeros((16,), _f32)
        return carry

    lax.fori_loop(0, RPT // 16, z1_body, 0)
    for q in range(RPT // ZR):
        pltpu.sync_copy(zbuf, acc.at[pl.ds(s * RPT + q * ZR, ZR)])
    pltpu.sync_copy(zbuf1, accx.at[pl.ds(s * RPT, RPT)])
    pltpu.sync_copy(zbuf1, accy.at[pl.ds(s * RPT, RPT)])
    pltpu.sync_copy(zbuf1, accz.at[pl.ds(s * RPT, RPT)])
    plsc.subcore_barrier()

    def do_chunk(base):
        base = pl.multiple_of(base, CE)
        pltpu.sync_copy(ii.at[pl.ds(base, CE)], idxv)
        pltpu.sync_copy(m.at[pl.ds(base, CE)], mbuf)
        pltpu.sync_copy(cu8.at[:, pl.ds(base, CE)], cbuf)
        pltpu.sync_copy(mbuf, acc.at[idxv], add=True)
        pltpu.sync_copy(cbuf.at[0], accx.at[idxv], add=True)
        pltpu.sync_copy(cbuf.at[1], accy.at[idxv], add=True)
        pltpu.sync_copy(cbuf.at[2], accz.at[idxv], add=True)

    def loop_body(t, carry):
        do_chunk(wid * EPW + t * CE)
        return carry

    lax.fori_loop(0, FULL, loop_body, 0)

    @pl.when(wid < TAIL_CHUNKS)
    def _():
        do_chunk(TAIL_BASE + wid * CE)

    plsc.subcore_barrier()
    pltpu.sync_copy(acc.at[pl.ds(s * RPT, RPT)],
                    parts.at[c, pl.ds(s * RPT, RPT)])

    @pl.when(s == 0)
    def _():
        pltpu.sync_copy(accx, cpx.at[c])
        pltpu.sync_copy(accy, cpy.at[c])
        pltpu.sync_copy(accz, cpz.at[c])


def _sc_scatter(m, cu8, ii):
    mesh = plsc.VectorSubcoreMesh(core_axis_name="c", subcore_axis_name="s")
    f = pl.kernel(
        _scatter_body,
        mesh=mesh,
        out_type=[
            jax.ShapeDtypeStruct((2, NPAD, H), _f32),
            jax.ShapeDtypeStruct((2, NPAD), _f32),
            jax.ShapeDtypeStruct((2, NPAD), _f32),
            jax.ShapeDtypeStruct((2, NPAD), _f32),
        ],
        scratch_types=[
            pltpu.VMEM((CE,), jnp.int32),
            pltpu.VMEM((CE, H), _f32),
            pltpu.VMEM((8, CE), _f32),
            pltpu.VMEM((ZR, H), _f32),
            pltpu.VMEM((RPT,), _f32),
            pltpu.MemorySpace.VMEM_SHARED((NPAD, H), _f32),
            pltpu.MemorySpace.VMEM_SHARED((NPAD,), _f32),
            pltpu.MemorySpace.VMEM_SHARED((NPAD,), _f32),
            pltpu.MemorySpace.VMEM_SHARED((NPAD,), _f32),
        ],
    )
    return f(m, cu8, ii)


def _node_body(h_ref, parts_ref, cpx_ref, cpy_ref, cpz_ref, pos3_ref,
               batch_ref, cond_ref,
               fw_ref, fb_ref, nw1a_ref, nw1b_ref, nb1_ref, nw2_ref, nb2_ref,
               lng_ref, lnb_ref, hnew_ref, posn_ref):
    agg = parts_ref[0] + parts_ref[1]
    hb = h_ref[...]
    hn = _silu(jnp.dot(hb, nw1a_ref[...], preferred_element_type=_f32)
               + jnp.dot(agg, nw1b_ref[...], preferred_element_type=_f32)
               + nb1_ref[...])
    h2 = jnp.dot(hn, nw2_ref[...], preferred_element_type=_f32) + nb2_ref[...]
    film = jnp.dot(cond_ref[...], fw_ref[...], preferred_element_type=_f32) + fb_ref[...]
    oh = (batch_ref[...] == lax.broadcasted_iota(jnp.int32, (BN, G), 1)).astype(_f32)
    gbt = jnp.dot(oh, film, preferred_element_type=_f32)
    h2 = gbt[:, :H] * h2 + gbt[:, H:]
    r = hb + h2
    mu = jnp.mean(r, axis=1, keepdims=True)
    var = jnp.mean((r - mu) * (r - mu), axis=1, keepdims=True)
    hnew_ref[...] = (r - mu) * lax.rsqrt(var + 1e-5) * lng_ref[...] + lnb_ref[...]
    cu = jnp.concatenate(
        [cpx_ref[0:1, :] + cpx_ref[1:2, :],
         cpy_ref[0:1, :] + cpy_ref[1:2, :],
         cpz_ref[0:1, :] + cpz_ref[1:2, :],
         jnp.zeros((1, NPAD), _f32)], axis=0)
    posn_ref[...] = pos3_ref[...] + cu


def _node_mlp(h, parts, cpx, cpy, cpz, pos3T, batch2, cond, fw, fb,
              nw1a, nw1b, nb1, nw2, nb2, lng, lnb):
    full = lambda i: (0, 0)
    return pl.pallas_call(
        _node_body,
        grid=(N // BN,),
        in_specs=[
            pl.BlockSpec((BN, H), lambda i: (i, 0)),
            pl.BlockSpec((2, BN, H), lambda i: (0, i, 0)),
            pl.BlockSpec((2, NPAD), full),
            pl.BlockSpec((2, NPAD), full),
            pl.BlockSpec((2, NPAD), full),
            pl.BlockSpec((4, NPAD), full),
            pl.BlockSpec((BN, 1), lambda i: (i, 0)),
            pl.BlockSpec((G, 128), full),
            pl.BlockSpec((128, 2 * H), full),
            pl.BlockSpec((1, 2 * H), full),
            pl.BlockSpec((H, H), full),
            pl.BlockSpec((H, H), full),
            pl.BlockSpec((1, H), full),
            pl.BlockSpec((H, H), full),
            pl.BlockSpec((1, H), full),
            pl.BlockSpec((1, H), full),
            pl.BlockSpec((1, H), full),
        ],
        out_specs=[
            pl.BlockSpec((BN, H), lambda i: (i, 0)),
            pl.BlockSpec((4, NPAD), lambda i: (0, 0)),
        ],
        out_shape=[
            jax.ShapeDtypeStruct((N, H), _f32),
            jax.ShapeDtypeStruct((4, NPAD), _f32),
        ],
    )(h, parts, cpx, cpy, cpz, pos3T, batch2, cond, fw, fb,
      nw1a, nw1b, nb1, nw2, nb2, lng, lnb)


def kernel(h, pos, edge_attr, cond, msg_w1, msg_b1, msg_w2, msg_b2,
           coord_w1, coord_b1, coord_w2, node_w1, node_b1, node_w2, node_b2,
           film_w, film_b, ln_g, ln_b, edge_index, batch):
    posT = jnp.pad(pos.T, ((0, 1), (0, NPAD - N)))
    pxh = posT[0]
    pyh = posT[1]
    pzh = posT[2]
    w1a = msg_w1[:H]
    w1b = msg_w1[H:2 * H]
    wd8 = jnp.zeros((8, H), _f32).at[3].set(msg_w1[2 * H])
    w1e = msg_w1[2 * H + 1:]
    ii = edge_index[0]
    jj = edge_index[1]

    ta, tb = _make_tables(h, w1a, w1b, msg_b1.reshape(1, H))
    ga, gb, rel8 = _sc_gather(ta, tb, pxh, pyh, pzh, ii, jj)
    m, cu8 = _edge_mlp(ga, gb, rel8, edge_attr, wd8, w1e, msg_w2,
                       msg_b2.reshape(1, H), coord_w1, coord_b1.reshape(1, H),
                       coord_w2.reshape(1, H))
    parts, cpx, cpy, cpz = _sc_scatter(m, cu8, ii)
    h_new, posnT = _node_mlp(
        h, parts, cpx, cpy, cpz, posT, batch.reshape(N, 1), cond, film_w,
        film_b.reshape(1, 2 * H), node_w1[:H], node_w1[H:],
        node_b1.reshape(1, H), node_w2, node_b2.reshape(1, H),
        ln_g.reshape(1, H), ln_b.reshape(1, H))
    return h_new, posnT[:3, :N].T


# speedup vs baseline: 6.1321x; 1.2848x over previous
import functools

import jax
import jax.numpy as jnp
from jax import lax
from jax.experimental import pallas as pl
from jax.experimental.pallas import tpu as pltpu
from jax.experimental.pallas import tpu_sc as plsc

N = 10000
E = 320000
H = 128
ED = 16
G = 64
CE = 128
NW = 32
FULL = (E // NW) // CE
EPW = FULL * CE
TAIL_BASE = NW * EPW
TAIL_CHUNKS = (E - TAIL_BASE) // CE
NG = CE // 16

BE = 2560
BN = 1000
NPAD = 10240
RPT = NPAD // 16
ZR = 128

_f32 = jnp.float32


def _silu(x):
    return x * jax.lax.logistic(x)


def _tables_body(h_ref, w1a_ref, w1b_ref, b1_ref, ta_ref, tb_ref):
    hb = h_ref[...]
    ta_ref[...] = jnp.dot(hb, w1a_ref[...], preferred_element_type=_f32) + b1_ref[...]
    tb_ref[...] = jnp.dot(hb, w1b_ref[...], preferred_element_type=_f32)


def _make_tables(h, w1a, w1b, b1):
    return pl.pallas_call(
        _tables_body,
        grid=(N // BN,),
        in_specs=[
            pl.BlockSpec((BN, H), lambda i: (i, 0)),
            pl.BlockSpec((H, H), lambda i: (0, 0)),
            pl.BlockSpec((H, H), lambda i: (0, 0)),
            pl.BlockSpec((1, H), lambda i: (0, 0)),
        ],
        out_specs=[
            pl.BlockSpec((BN, H), lambda i: (i, 0)),
            pl.BlockSpec((BN, H), lambda i: (i, 0)),
        ],
        out_shape=[
            jax.ShapeDtypeStruct((N, H), _f32),
            jax.ShapeDtypeStruct((N, H), _f32),
        ],
    )(h, w1a, w1b, b1)


def _gather_body(ta, tb, pxh, pyh, pzh, ii, jj, g, rel8, *scr):
    c = lax.axis_index("c")
    s = lax.axis_index("s")
    wid = s * 2 + c
    S0 = scr[:12]
    S1 = scr[12:24]
    nch = FULL + jnp.where(wid < TAIL_CHUNKS, 1, 0)

    def base_of(t):
        return pl.multiple_of(
            jnp.where(t < FULL, wid * EPW + t * CE, TAIL_BASE + wid * CE), CE)

    for S in (S0, S1):
        bufr = S[4]
        for d in range(4, 8):
            for gg in range(NG):
                bufr[d, pl.ds(gg * 16, 16)] = jnp.zeros((16,), _f32)

    def gather_copies(S):
        idxi, idxj, bufa, bufb, bufr, pxi, pyi, pzi, pxj, pyj, pzj, sg = S[:12]
        return [pltpu.make_async_copy(ta.at[idxi], bufa, sg),
                pltpu.make_async_copy(tb.at[idxj], bufb, sg),
                pltpu.make_async_copy(pxh.at[idxi], pxi, sg),
                pltpu.make_async_copy(pyh.at[idxi], pyi, sg),
                pltpu.make_async_copy(pzh.at[idxi], pzi, sg),
                pltpu.make_async_copy(pxh.at[idxj], pxj, sg),
                pltpu.make_async_copy(pyh.at[idxj], pyj, sg),
                pltpu.make_async_copy(pzh.at[idxj], pzj, sg)]

    def start_g(S, base, sw):
        pltpu.sync_copy(ii.at[pl.ds(base, CE)], S[0])
        pltpu.sync_copy(jj.at[pl.ds(base, CE)], S[1])
        for cp in gather_copies(S):
            cp.start()

    def wait_g(S):
        for cp in gather_copies(S):
            cp.wait()

    def write_copies(S, base, sw):
        bufa, bufr = S[2], S[4]
        return [pltpu.make_async_copy(bufa, g.at[pl.ds(base, CE)], sw),
                pltpu.make_async_copy(bufr, rel8.at[:, pl.ds(base, CE)], sw)]

    def compute(S):
        idxi, idxj, bufa, bufb, bufr, pxi, pyi, pzi, pxj, pyj, pzj, sg = S[:12]

        def esum(e, carry):
            for k in range(H // 16):
                dk = pl.ds(k * 16, 16)
                bufa[e, dk] = bufa[e, dk] + bufb[e, dk]
            return carry

        lax.fori_loop(0, CE, esum, 0)
        for gg in range(NG):
            d16 = pl.ds(gg * 16, 16)
            rx = pxi[d16] - pxj[d16]
            ry = pyi[d16] - pyj[d16]
            rz = pzi[d16] - pzj[d16]
            bufr[0, d16] = rx
            bufr[1, d16] = ry
            bufr[2, d16] = rz
            bufr[3, d16] = rx * rx + ry * ry + rz * rz

    sw0, sw1 = scr[24], scr[25]
    start_g(S0, base_of(0), sw0)

    def pair(p, carry):
        t0 = 2 * p
        wait_g(S0)

        @pl.when(p > 0)
        def _():
            for cp in write_copies(S1, base_of(t0 - 1), sw1):
                cp.wait()

        start_g(S1, base_of(t0 + 1), sw1)
        compute(S0)
        for cp in write_copies(S0, base_of(t0), sw0):
            cp.start()
        wait_g(S1)
        for cp in write_copies(S0, base_of(t0), sw0):
            cp.wait()

        @pl.when(t0 + 2 < nch)
        def _():
            start_g(S0, base_of(t0 + 2), sw0)

        compute(S1)
        for cp in write_copies(S1, base_of(t0 + 1), sw1):
            cp.start()
        return carry

    lax.fori_loop(0, FULL // 2, pair, 0)

    @pl.when(nch > FULL)
    def _():
        wait_g(S0)
        for cp in write_copies(S1, base_of(FULL - 1), sw1):
            cp.wait()
        compute(S0)
        for cp in write_copies(S0, base_of(FULL), sw0):
            cp.start()
        for cp in write_copies(S0, base_of(FULL), sw0):
            cp.wait()

    @pl.when(nch == FULL)
    def _():
        for cp in write_copies(S1, base_of(FULL - 1), sw1):
            cp.wait()


def _sc_gather(ta, tb, pxh, pyh, pzh, ii, jj):
    mesh = plsc.VectorSubcoreMesh(core_axis_name="c", subcore_axis_name="s")
    bufset = [
        pltpu.VMEM((CE,), jnp.int32),
        pltpu.VMEM((CE,), jnp.int32),
        pltpu.VMEM((CE, H), _f32),
        pltpu.VMEM((CE, H), _f32),
        pltpu.VMEM((8, CE), _f32),
        pltpu.VMEM((CE,), _f32),
        pltpu.VMEM((CE,), _f32),
        pltpu.VMEM((CE,), _f32),
        pltpu.VMEM((CE,), _f32),
        pltpu.VMEM((CE,), _f32),
        pltpu.VMEM((CE,), _f32),
        pltpu.SemaphoreType.DMA,
    ]
    f = pl.kernel(
        _gather_body,
        mesh=mesh,
        out_type=[
            jax.ShapeDtypeStruct((E, H), _f32),
            jax.ShapeDtypeStruct((8, E), _f32),
        ],
        scratch_types=bufset + bufset + [pltpu.SemaphoreType.DMA,
                                         pltpu.SemaphoreType.DMA],
    )
    return f(ta, tb, pxh, pyh, pzh, ii, jj)


def _edge_body(g_ref, r8_ref, ea_ref, wd8_ref, w1e_ref, w2_ref,
               b2_ref, cw1_ref, cb1_ref, cw2_ref, m_ref, cu_ref):
    r8 = r8_ref[...]
    pre = (g_ref[...]
           + jnp.dot(ea_ref[...], w1e_ref[...], preferred_element_type=_f32)
           + lax.dot_general(r8, wd8_ref[...], (((0,), (0,)), ((), ())),
                             preferred_element_type=_f32))
    m1 = _silu(pre)
    m2 = _silu(jnp.dot(m1, w2_ref[...], preferred_element_type=_f32) + b2_ref[...])
    t = _silu(jnp.dot(m2, cw1_ref[...], preferred_element_type=_f32) + cb1_ref[...])
    cw_row = lax.dot_general(cw2_ref[...], t, (((1,), (1,)), ((), ())),
                             preferred_element_type=_f32)
    dsq_row = r8[3:4, :]
    scale = cw_row * lax.rsqrt(dsq_row + 1e-8)
    cu = r8[0:3, :] * scale
    m_ref[...] = m2
    cu_ref[...] = jnp.concatenate(
        [cu, jnp.zeros((5, cu.shape[1]), _f32)], axis=0)


def _edge_mlp(g, rel8, ea, wd8, w1e, w2, b2, cw1, cb1, cw2):
    full = lambda i: (0, 0)
    return pl.pallas_call(
        _edge_body,
        grid=(E // BE,),
        in_specs=[
            pl.BlockSpec((BE, H), lambda i: (i, 0)),
            pl.BlockSpec((8, BE), lambda i: (0, i)),
            pl.BlockSpec((BE, ED), lambda i: (i, 0)),
            pl.BlockSpec((8, H), full),
            pl.BlockSpec((ED, H), full),
            pl.BlockSpec((H, H), full),
            pl.BlockSpec((1, H), full),
            pl.BlockSpec((H, H), full),
            pl.BlockSpec((1, H), full),
            pl.BlockSpec((1, H), full),
        ],
        out_specs=[
            pl.BlockSpec((BE, H), lambda i: (i, 0)),
            pl.BlockSpec((8, BE), lambda i: (0, i)),
        ],
        out_shape=[
            jax.ShapeDtypeStruct((E, H), _f32),
            jax.ShapeDtypeStruct((8, E), _f32),
        ],
    )(g, rel8, ea, wd8, w1e, w2, b2, cw1, cb1, cw2)


def _scatter_body(m, cu8, ii, parts, cpx, cpy, cpz,
                  idxv, mbuf, cbuf, sl0, idxv1, mbuf1, cbuf1, sl1,
                  zbuf1, acc, accx, accy, accz):
    c = lax.axis_index("c")
    s = lax.axis_index("s")
    wid = s * 2 + c

    def zb_body(r, carry):
        for k in range(H // 16):
            mbuf[r, pl.ds(k * 16, 16)] = jnp.zeros((16,), _f32)
        return carry

    lax.fori_loop(0, CE, zb_body, 0)

    def z1_body(r, carry):
        zbuf1[pl.ds(r * 16, 16)] = jnp.zeros((16,), _f32)
        return carry

    lax.fori_loop(0, RPT // 16, z1_body, 0)
    for q in range(RPT // CE):
        pltpu.sync_copy(mbuf, acc.at[pl.ds(s * RPT + q * CE, CE)])
    pltpu.sync_copy(zbuf1, accx.at[pl.ds(s * RPT, RPT)])
    pltpu.sync_copy(zbuf1, accy.at[pl.ds(s * RPT, RPT)])
    pltpu.sync_copy(zbuf1, accz.at[pl.ds(s * RPT, RPT)])
    plsc.subcore_barrier()

    nch = FULL + jnp.where(wid < TAIL_CHUNKS, 1, 0)

    def base_of(t):
        return pl.multiple_of(
            jnp.where(t < FULL, wid * EPW + t * CE, TAIL_BASE + wid * CE), CE)

    def load_copies(S, base):
        idxv, mbuf, cbuf, sl = S
        return [pltpu.make_async_copy(ii.at[pl.ds(base, CE)], idxv, sl),
                pltpu.make_async_copy(m.at[pl.ds(base, CE)], mbuf, sl),
                pltpu.make_async_copy(cu8.at[:, pl.ds(base, CE)], cbuf, sl)]

    def scatter4(S):
        idxv, mbuf, cbuf, sl = S
        pltpu.sync_copy(mbuf, acc.at[idxv], add=True)
        pltpu.sync_copy(cbuf.at[0], accx.at[idxv], add=True)
        pltpu.sync_copy(cbuf.at[1], accy.at[idxv], add=True)
        pltpu.sync_copy(cbuf.at[2], accz.at[idxv], add=True)

    S0 = (idxv, mbuf, cbuf, sl0)
    S1 = (idxv1, mbuf1, cbuf1, sl1)
    for cp in load_copies(S0, base_of(0)):
        cp.start()

    def pair(p, carry):
        t0 = 2 * p
        for cp in load_copies(S0, base_of(t0)):
            cp.wait()
        for cp in load_copies(S1, base_of(t0 + 1)):
            cp.start()
        scatter4(S0)
        for cp in load_copies(S1, base_of(t0 + 1)):
            cp.wait()

        @pl.when(t0 + 2 < nch)
        def _():
            for cp in load_copies(S0, base_of(t0 + 2)):
                cp.start()

        scatter4(S1)
        return carry

    lax.fori_loop(0, FULL // 2, pair, 0)

    @pl.when(nch > FULL)
    def _():
        for cp in load_copies(S0, base_of(FULL)):
            cp.wait()
        scatter4(S0)

    plsc.subcore_barrier()
    pltpu.sync_copy(acc.at[pl.ds(s * RPT, RPT)],
                    parts.at[c, pl.ds(s * RPT, RPT)])

    @pl.when(s == 0)
    def _():
        pltpu.sync_copy(accx, cpx.at[c])
        pltpu.sync_copy(accy, cpy.at[c])
        pltpu.sync_copy(accz, cpz.at[c])


def _sc_scatter(m, cu8, ii):
    mesh = plsc.VectorSubcoreMesh(core_axis_name="c", subcore_axis_name="s")
    f = pl.kernel(
        _scatter_body,
        mesh=mesh,
        out_type=[
            jax.ShapeDtypeStruct((2, NPAD, H), _f32),
            jax.ShapeDtypeStruct((2, NPAD), _f32),
            jax.ShapeDtypeStruct((2, NPAD), _f32),
            jax.ShapeDtypeStruct((2, NPAD), _f32),
        ],
        scratch_types=[
            pltpu.VMEM((CE,), jnp.int32),
            pltpu.VMEM((CE, H), _f32),
            pltpu.VMEM((8, CE), _f32),
            pltpu.SemaphoreType.DMA,
            pltpu.VMEM((CE,), jnp.int32),
            pltpu.VMEM((CE, H), _f32),
            pltpu.VMEM((8, CE), _f32),
            pltpu.SemaphoreType.DMA,
            pltpu.VMEM((RPT,), _f32),
            pltpu.MemorySpace.VMEM_SHARED((NPAD, H), _f32),
            pltpu.MemorySpace.VMEM_SHARED((NPAD,), _f32),
            pltpu.MemorySpace.VMEM_SHARED((NPAD,), _f32),
            pltpu.MemorySpace.VMEM_SHARED((NPAD,), _f32),
        ],
    )
    return f(m, cu8, ii)


def _node_body(h_ref, parts_ref, cpx_ref, cpy_ref, cpz_ref, pos3_ref,
               batch_ref, cond_ref,
               fw_ref, fb_ref, nw1a_ref, nw1b_ref, nb1_ref, nw2_ref, nb2_ref,
               lng_ref, lnb_ref, hnew_ref, posn_ref):
    agg = parts_ref[0] + parts_ref[1]
    hb = h_ref[...]
    hn = _silu(jnp.dot(hb, nw1a_ref[...], preferred_element_type=_f32)
               + jnp.dot(agg, nw1b_ref[...], preferred_element_type=_f32)
               + nb1_ref[...])
    h2 = jnp.dot(hn, nw2_ref[...], preferred_element_type=_f32) + nb2_ref[...]
    film = jnp.dot(cond_ref[...], fw_ref[...], preferred_element_type=_f32) + fb_ref[...]
    oh = (batch_ref[...] == lax.broadcasted_iota(jnp.int32, (BN, G), 1)).astype(_f32)
    gbt = jnp.dot(oh, film, preferred_element_type=_f32)
    h2 = gbt[:, :H] * h2 + gbt[:, H:]
    r = hb + h2
    mu = jnp.mean(r, axis=1, keepdims=True)
    var = jnp.mean((r - mu) * (r - mu), axis=1, keepdims=True)
    hnew_ref[...] = (r - mu) * lax.rsqrt(var + 1e-5) * lng_ref[...] + lnb_ref[...]
    cu = jnp.concatenate(
        [cpx_ref[0:1, :] + cpx_ref[1:2, :],
         cpy_ref[0:1, :] + cpy_ref[1:2, :],
         cpz_ref[0:1, :] + cpz_ref[1:2, :],
         jnp.zeros((1, NPAD), _f32)], axis=0)
    posn_ref[...] = pos3_ref[...] + cu


def _node_mlp(h, parts, cpx, cpy, cpz, pos3T, batch2, cond, fw, fb,
              nw1a, nw1b, nb1, nw2, nb2, lng, lnb):
    full = lambda i: (0, 0)
    return pl.pallas_call(
        _node_body,
        grid=(N // BN,),
        in_specs=[
            pl.BlockSpec((BN, H), lambda i: (i, 0)),
            pl.BlockSpec((2, BN, H), lambda i: (0, i, 0)),
            pl.BlockSpec((2, NPAD), full),
            pl.BlockSpec((2, NPAD), full),
            pl.BlockSpec((2, NPAD), full),
            pl.BlockSpec((4, NPAD), full),
            pl.BlockSpec((BN, 1), lambda i: (i, 0)),
            pl.BlockSpec((G, 128), full),
            pl.BlockSpec((128, 2 * H), full),
            pl.BlockSpec((1, 2 * H), full),
            pl.BlockSpec((H, H), full),
            pl.BlockSpec((H, H), full),
            pl.BlockSpec((1, H), full),
            pl.BlockSpec((H, H), full),
            pl.BlockSpec((1, H), full),
            pl.BlockSpec((1, H), full),
            pl.BlockSpec((1, H), full),
        ],
        out_specs=[
            pl.BlockSpec((BN, H), lambda i: (i, 0)),
            pl.BlockSpec((4, NPAD), lambda i: (0, 0)),
        ],
        out_shape=[
            jax.ShapeDtypeStruct((N, H), _f32),
            jax.ShapeDtypeStruct((4, NPAD), _f32),
        ],
    )(h, parts, cpx, cpy, cpz, pos3T, batch2, cond, fw, fb,
      nw1a, nw1b, nb1, nw2, nb2, lng, lnb)


def kernel(h, pos, edge_attr, cond, msg_w1, msg_b1, msg_w2, msg_b2,
           coord_w1, coord_b1, coord_w2, node_w1, node_b1, node_w2, node_b2,
           film_w, film_b, ln_g, ln_b, edge_index, batch):
    posT = jnp.pad(pos.T, ((0, 1), (0, NPAD - N)))
    pxh = posT[0]
    pyh = posT[1]
    pzh = posT[2]
    w1a = msg_w1[:H]
    w1b = msg_w1[H:2 * H]
    wd8 = jnp.zeros((8, H), _f32).at[3].set(msg_w1[2 * H])
    w1e = msg_w1[2 * H + 1:]
    ii = edge_index[0]
    jj = edge_index[1]

    ta, tb = _make_tables(h, w1a, w1b, msg_b1.reshape(1, H))
    g, rel8 = _sc_gather(ta, tb, pxh, pyh, pzh, ii, jj)
    m, cu8 = _edge_mlp(g, rel8, edge_attr, wd8, w1e, msg_w2,
                       msg_b2.reshape(1, H), coord_w1, coord_b1.reshape(1, H),
                       coord_w2.reshape(1, H))
    parts, cpx, cpy, cpz = _sc_scatter(m, cu8, ii)
    h_new, posnT = _node_mlp(
        h, parts, cpx, cpy, cpz, posT, batch.reshape(N, 1), cond, film_w,
        film_b.reshape(1, 2 * H), node_w1[:H], node_w1[H:],
        node_b1.reshape(1, H), node_w2, node_b2.reshape(1, H),
        ln_g.reshape(1, H), ln_b.reshape(1, H))
    return h_new, posnT[:3, :N].T


# speedup vs baseline: 6.5179x; 1.0629x over previous
import functools

import jax
import jax.numpy as jnp
from jax import lax
from jax.experimental import pallas as pl
from jax.experimental.pallas import tpu as pltpu
from jax.experimental.pallas import tpu_sc as plsc

N = 10000
E = 320000
H = 128
ED = 16
G = 64
CE = 128
NW = 32
FULL = (E // NW) // CE
EPW = FULL * CE
TAIL_BASE = NW * EPW
TAIL_CHUNKS = (E - TAIL_BASE) // CE
NG = CE // 16

BE = 2560
BN = 1000
NPAD = 10240
RPT = NPAD // 16
ZR = 128

_f32 = jnp.float32


def _silu(x):
    return x * jax.lax.logistic(x)


def _tables_body(h_ref, w1a_ref, w1b_ref, b1_ref, ta_ref, tb_ref):
    hb = h_ref[...]
    ta_ref[...] = jnp.dot(hb, w1a_ref[...], preferred_element_type=_f32) + b1_ref[...]
    tb_ref[...] = jnp.dot(hb, w1b_ref[...], preferred_element_type=_f32)


def _make_tables(h, w1a, w1b, b1):
    return pl.pallas_call(
        _tables_body,
        grid=(N // BN,),
        in_specs=[
            pl.BlockSpec((BN, H), lambda i: (i, 0)),
            pl.BlockSpec((H, H), lambda i: (0, 0)),
            pl.BlockSpec((H, H), lambda i: (0, 0)),
            pl.BlockSpec((1, H), lambda i: (0, 0)),
        ],
        out_specs=[
            pl.BlockSpec((BN, H), lambda i: (i, 0)),
            pl.BlockSpec((BN, H), lambda i: (i, 0)),
        ],
        out_shape=[
            jax.ShapeDtypeStruct((N, H), _f32),
            jax.ShapeDtypeStruct((N, H), _f32),
        ],
    )(h, w1a, w1b, b1)


def _gather_body(ta, tb, pxh, pyh, pzh, ii, jj, g, rel8, *scr):
    c = lax.axis_index("c")
    s = lax.axis_index("s")
    wid = s * 2 + c
    S0 = scr[:13]
    S1 = scr[13:26]
    nch = FULL + jnp.where(wid < TAIL_CHUNKS, 1, 0)

    def base_of(t):
        return pl.multiple_of(
            jnp.where(t < FULL, wid * EPW + t * CE, TAIL_BASE + wid * CE), CE)

    for S in (S0, S1):
        bufr = S[4]
        for d in range(4, 8):
            for gg in range(NG):
                bufr[d, pl.ds(gg * 16, 16)] = jnp.zeros((16,), _f32)

    def gather_copies(S):
        idxi, idxj, bufa, bufb, bufr, pxi, pyi, pzi, pxj, pyj, pzj, sg = S[:12]
        return [pltpu.make_async_copy(ta.at[idxi], bufa, sg),
                pltpu.make_async_copy(tb.at[idxj], bufb, sg),
                pltpu.make_async_copy(pxh.at[idxi], pxi, sg),
                pltpu.make_async_copy(pyh.at[idxi], pyi, sg),
                pltpu.make_async_copy(pzh.at[idxi], pzi, sg),
                pltpu.make_async_copy(pxh.at[idxj], pxj, sg),
                pltpu.make_async_copy(pyh.at[idxj], pyj, sg),
                pltpu.make_async_copy(pzh.at[idxj], pzj, sg)]

    def idx_copies(S, base):
        return [pltpu.make_async_copy(ii.at[pl.ds(base, CE)], S[0], S[12]),
                pltpu.make_async_copy(jj.at[pl.ds(base, CE)], S[1], S[12])]

    def start_idx(S, base):
        for cp in idx_copies(S, base):
            cp.start()

    def start_g(S, base, sw):
        for cp in idx_copies(S, base):
            cp.wait()
        for cp in gather_copies(S):
            cp.start()

    def wait_g(S):
        for cp in gather_copies(S):
            cp.wait()

    def write_copies(S, base, sw):
        bufa, bufr = S[2], S[4]
        return [pltpu.make_async_copy(bufa, g.at[pl.ds(base, CE)], sw),
                pltpu.make_async_copy(bufr, rel8.at[:, pl.ds(base, CE)], sw)]

    def compute(S):
        idxi, idxj, bufa, bufb, bufr, pxi, pyi, pzi, pxj, pyj, pzj, sg = S[:12]

        def esum(e, carry):
            for k in range(H // 16):
                dk = pl.ds(k * 16, 16)
                bufa[e, dk] = bufa[e, dk] + bufb[e, dk]
            return carry

        lax.fori_loop(0, CE, esum, 0)
        for gg in range(NG):
            d16 = pl.ds(gg * 16, 16)
            rx = pxi[d16] - pxj[d16]
            ry = pyi[d16] - pyj[d16]
            rz = pzi[d16] - pzj[d16]
            bufr[0, d16] = rx
            bufr[1, d16] = ry
            bufr[2, d16] = rz
            bufr[3, d16] = rx * rx + ry * ry + rz * rz

    sw0, sw1 = scr[26], scr[27]
    start_idx(S0, base_of(0))
    start_g(S0, base_of(0), sw0)
    start_idx(S1, base_of(1))

    def pair(p, carry):
        t0 = 2 * p
        wait_g(S0)

        @pl.when(p > 0)
        def _():
            for cp in write_copies(S1, base_of(t0 - 1), sw1):
                cp.wait()

        start_g(S1, base_of(t0 + 1), sw1)

        @pl.when(t0 + 2 < nch)
        def _():
            start_idx(S0, base_of(t0 + 2))

        compute(S0)
        for cp in write_copies(S0, base_of(t0), sw0):
            cp.start()
        wait_g(S1)
        for cp in write_copies(S0, base_of(t0), sw0):
            cp.wait()

        @pl.when(t0 + 2 < nch)
        def _():
            start_g(S0, base_of(t0 + 2), sw0)

        @pl.when(t0 + 3 < nch)
        def _():
            start_idx(S1, base_of(t0 + 3))

        compute(S1)
        for cp in write_copies(S1, base_of(t0 + 1), sw1):
            cp.start()
        return carry

    lax.fori_loop(0, FULL // 2, pair, 0)

    @pl.when(nch > FULL)
    def _():
        wait_g(S0)
        for cp in write_copies(S1, base_of(FULL - 1), sw1):
            cp.wait()
        compute(S0)
        for cp in write_copies(S0, base_of(FULL), sw0):
            cp.start()
        for cp in write_copies(S0, base_of(FULL), sw0):
            cp.wait()

    @pl.when(nch == FULL)
    def _():
        for cp in write_copies(S1, base_of(FULL - 1), sw1):
            cp.wait()


def _sc_gather(ta, tb, pxh, pyh, pzh, ii, jj):
    mesh = plsc.VectorSubcoreMesh(core_axis_name="c", subcore_axis_name="s")
    bufset = [
        pltpu.VMEM((CE,), jnp.int32),
        pltpu.VMEM((CE,), jnp.int32),
        pltpu.VMEM((CE, H), _f32),
        pltpu.VMEM((CE, H), _f32),
        pltpu.VMEM((8, CE), _f32),
        pltpu.VMEM((CE,), _f32),
        pltpu.VMEM((CE,), _f32),
        pltpu.VMEM((CE,), _f32),
        pltpu.VMEM((CE,), _f32),
        pltpu.VMEM((CE,), _f32),
        pltpu.VMEM((CE,), _f32),
        pltpu.SemaphoreType.DMA,
        pltpu.SemaphoreType.DMA,
    ]
    f = pl.kernel(
        _gather_body,
        mesh=mesh,
        out_type=[
            jax.ShapeDtypeStruct((E, H), _f32),
            jax.ShapeDtypeStruct((8, E), _f32),
        ],
        scratch_types=bufset + bufset + [pltpu.SemaphoreType.DMA,
                                         pltpu.SemaphoreType.DMA],
    )
    return f(ta, tb, pxh, pyh, pzh, ii, jj)


def _edge_body(g_ref, r8_ref, ea_ref, wd8_ref, w1e_ref, w2_ref,
               b2_ref, cw1_ref, cb1_ref, cw2_ref, m_ref, cu_ref):
    r8 = r8_ref[...]
    pre = (g_ref[...]
           + jnp.dot(ea_ref[...], w1e_ref[...], preferred_element_type=_f32)
           + lax.dot_general(r8, wd8_ref[...], (((0,), (0,)), ((), ())),
                             preferred_element_type=_f32))
    m1 = _silu(pre)
    m2 = _silu(jnp.dot(m1, w2_ref[...], preferred_element_type=_f32) + b2_ref[...])
    t = _silu(jnp.dot(m2, cw1_ref[...], preferred_element_type=_f32) + cb1_ref[...])
    cw_row = lax.dot_general(cw2_ref[...], t, (((1,), (1,)), ((), ())),
                             preferred_element_type=_f32)
    dsq_row = r8[3:4, :]
    scale = cw_row * lax.rsqrt(dsq_row + 1e-8)
    cu = r8[0:3, :] * scale
    m_ref[...] = m2
    cu_ref[...] = jnp.concatenate(
        [cu, jnp.zeros((5, cu.shape[1]), _f32)], axis=0)


def _edge_mlp(g, rel8, ea, wd8, w1e, w2, b2, cw1, cb1, cw2):
    full = lambda i: (0, 0)
    return pl.pallas_call(
        _edge_body,
        grid=(E // BE,),
        in_specs=[
            pl.BlockSpec((BE, H), lambda i: (i, 0)),
            pl.BlockSpec((8, BE), lambda i: (0, i)),
            pl.BlockSpec((BE, ED), lambda i: (i, 0)),
            pl.BlockSpec((8, H), full),
            pl.BlockSpec((ED, H), full),
            pl.BlockSpec((H, H), full),
            pl.BlockSpec((1, H), full),
            pl.BlockSpec((H, H), full),
            pl.BlockSpec((1, H), full),
            pl.BlockSpec((1, H), full),
        ],
        out_specs=[
            pl.BlockSpec((BE, H), lambda i: (i, 0)),
            pl.BlockSpec((8, BE), lambda i: (0, i)),
        ],
        out_shape=[
            jax.ShapeDtypeStruct((E, H), _f32),
            jax.ShapeDtypeStruct((8, E), _f32),
        ],
    )(g, rel8, ea, wd8, w1e, w2, b2, cw1, cb1, cw2)


def _scatter_body(m, cu8, ii, parts, cpx, cpy, cpz,
                  idxv, mbuf, cbuf, sl0, idxv1, mbuf1, cbuf1, sl1,
                  zbuf1, acc, accx, accy, accz):
    c = lax.axis_index("c")
    s = lax.axis_index("s")
    wid = s * 2 + c

    def zb_body(r, carry):
        for k in range(H // 16):
            mbuf[r, pl.ds(k * 16, 16)] = jnp.zeros((16,), _f32)
        return carry

    lax.fori_loop(0, CE, zb_body, 0)

    def z1_body(r, carry):
        zbuf1[pl.ds(r * 16, 16)] = jnp.zeros((16,), _f32)
        return carry

    lax.fori_loop(0, RPT // 16, z1_body, 0)
    for q in range(RPT // CE):
        pltpu.sync_copy(mbuf, acc.at[pl.ds(s * RPT + q * CE, CE)])
    pltpu.sync_copy(zbuf1, accx.at[pl.ds(s * RPT, RPT)])
    pltpu.sync_copy(zbuf1, accy.at[pl.ds(s * RPT, RPT)])
    pltpu.sync_copy(zbuf1, accz.at[pl.ds(s * RPT, RPT)])
    plsc.subcore_barrier()

    nch = FULL + jnp.where(wid < TAIL_CHUNKS, 1, 0)

    def base_of(t):
        return pl.multiple_of(
            jnp.where(t < FULL, wid * EPW + t * CE, TAIL_BASE + wid * CE), CE)

    def load_copies(S, base):
        idxv, mbuf, cbuf, sl = S
        return [pltpu.make_async_copy(ii.at[pl.ds(base, CE)], idxv, sl),
                pltpu.make_async_copy(m.at[pl.ds(base, CE)], mbuf, sl),
                pltpu.make_async_copy(cu8.at[:, pl.ds(base, CE)], cbuf, sl)]

    def scatter4(S):
        idxv, mbuf, cbuf, sl = S
        pltpu.sync_copy(mbuf, acc.at[idxv], add=True)
        pltpu.sync_copy(cbuf.at[0], accx.at[idxv], add=True)
        pltpu.sync_copy(cbuf.at[1], accy.at[idxv], add=True)
        pltpu.sync_copy(cbuf.at[2], accz.at[idxv], add=True)

    S0 = (idxv, mbuf, cbuf, sl0)
    S1 = (idxv1, mbuf1, cbuf1, sl1)
    for cp in load_copies(S0, base_of(0)):
        cp.start()

    def pair(p, carry):
        t0 = 2 * p
        for cp in load_copies(S0, base_of(t0)):
            cp.wait()
        for cp in load_copies(S1, base_of(t0 + 1)):
            cp.start()
        scatter4(S0)
        for cp in load_copies(S1, base_of(t0 + 1)):
            cp.wait()

        @pl.when(t0 + 2 < nch)
        def _():
            for cp in load_copies(S0, base_of(t0 + 2)):
                cp.start()

        scatter4(S1)
        return carry

    lax.fori_loop(0, FULL // 2, pair, 0)

    @pl.when(nch > FULL)
    def _():
        for cp in load_copies(S0, base_of(FULL)):
            cp.wait()
        scatter4(S0)

    plsc.subcore_barrier()
    pltpu.sync_copy(acc.at[pl.ds(s * RPT, RPT)],
                    parts.at[c, pl.ds(s * RPT, RPT)])

    @pl.when(s == 0)
    def _():
        pltpu.sync_copy(accx, cpx.at[c])
        pltpu.sync_copy(accy, cpy.at[c])
        pltpu.sync_copy(accz, cpz.at[c])


def _sc_scatter(m, cu8, ii):
    mesh = plsc.VectorSubcoreMesh(core_axis_name="c", subcore_axis_name="s")
    f = pl.kernel(
        _scatter_body,
        mesh=mesh,
        out_type=[
            jax.ShapeDtypeStruct((2, NPAD, H), _f32),
            jax.ShapeDtypeStruct((2, NPAD), _f32),
            jax.ShapeDtypeStruct((2, NPAD), _f32),
            jax.ShapeDtypeStruct((2, NPAD), _f32),
        ],
        scratch_types=[
            pltpu.VMEM((CE,), jnp.int32),
            pltpu.VMEM((CE, H), _f32),
            pltpu.VMEM((8, CE), _f32),
            pltpu.SemaphoreType.DMA,
            pltpu.VMEM((CE,), jnp.int32),
            pltpu.VMEM((CE, H), _f32),
            pltpu.VMEM((8, CE), _f32),
            pltpu.SemaphoreType.DMA,
            pltpu.VMEM((RPT,), _f32),
            pltpu.MemorySpace.VMEM_SHARED((NPAD, H), _f32),
            pltpu.MemorySpace.VMEM_SHARED((NPAD,), _f32),
            pltpu.MemorySpace.VMEM_SHARED((NPAD,), _f32),
            pltpu.MemorySpace.VMEM_SHARED((NPAD,), _f32),
        ],
    )
    return f(m, cu8, ii)


def _node_body(h_ref, parts_ref, cpx_ref, cpy_ref, cpz_ref, pos3_ref,
               batch_ref, cond_ref,
               fw_ref, fb_ref, nw1a_ref, nw1b_ref, nb1_ref, nw2_ref, nb2_ref,
               lng_ref, lnb_ref, hnew_ref, posn_ref):
    agg = parts_ref[0] + parts_ref[1]
    hb = h_ref[...]
    hn = _silu(jnp.dot(hb, nw1a_ref[...], preferred_element_type=_f32)
               + jnp.dot(agg, nw1b_ref[...], preferred_element_type=_f32)
               + nb1_ref[...])
    h2 = jnp.dot(hn, nw2_ref[...], preferred_element_type=_f32) + nb2_ref[...]
    film = jnp.dot(cond_ref[...], fw_ref[...], preferred_element_type=_f32) + fb_ref[...]
    oh = (batch_ref[...] == lax.broadcasted_iota(jnp.int32, (BN, G), 1)).astype(_f32)
    gbt = jnp.dot(oh, film, preferred_element_type=_f32)
    h2 = gbt[:, :H] * h2 + gbt[:, H:]
    r = hb + h2
    mu = jnp.mean(r, axis=1, keepdims=True)
    var = jnp.mean((r - mu) * (r - mu), axis=1, keepdims=True)
    hnew_ref[...] = (r - mu) * lax.rsqrt(var + 1e-5) * lng_ref[...] + lnb_ref[...]
    cu = jnp.concatenate(
        [cpx_ref[0:1, :] + cpx_ref[1:2, :],
         cpy_ref[0:1, :] + cpy_ref[1:2, :],
         cpz_ref[0:1, :] + cpz_ref[1:2, :],
         jnp.zeros((1, NPAD), _f32)], axis=0)
    posn_ref[...] = pos3_ref[...] + cu


def _node_mlp(h, parts, cpx, cpy, cpz, pos3T, batch2, cond, fw, fb,
              nw1a, nw1b, nb1, nw2, nb2, lng, lnb):
    full = lambda i: (0, 0)
    return pl.pallas_call(
        _node_body,
        grid=(N // BN,),
        in_specs=[
            pl.BlockSpec((BN, H), lambda i: (i, 0)),
            pl.BlockSpec((2, BN, H), lambda i: (0, i, 0)),
            pl.BlockSpec((2, NPAD), full),
            pl.BlockSpec((2, NPAD), full),
            pl.BlockSpec((2, NPAD), full),
            pl.BlockSpec((4, NPAD), full),
            pl.BlockSpec((BN, 1), lambda i: (i, 0)),
            pl.BlockSpec((G, 128), full),
            pl.BlockSpec((128, 2 * H), full),
            pl.BlockSpec((1, 2 * H), full),
            pl.BlockSpec((H, H), full),
            pl.BlockSpec((H, H), full),
            pl.BlockSpec((1, H), full),
            pl.BlockSpec((H, H), full),
            pl.BlockSpec((1, H), full),
            pl.BlockSpec((1, H), full),
            pl.BlockSpec((1, H), full),
        ],
        out_specs=[
            pl.BlockSpec((BN, H), lambda i: (i, 0)),
            pl.BlockSpec((4, NPAD), lambda i: (0, 0)),
        ],
        out_shape=[
            jax.ShapeDtypeStruct((N, H), _f32),
            jax.ShapeDtypeStruct((4, NPAD), _f32),
        ],
    )(h, parts, cpx, cpy, cpz, pos3T, batch2, cond, fw, fb,
      nw1a, nw1b, nb1, nw2, nb2, lng, lnb)


def kernel(h, pos, edge_attr, cond, msg_w1, msg_b1, msg_w2, msg_b2,
           coord_w1, coord_b1, coord_w2, node_w1, node_b1, node_w2, node_b2,
           film_w, film_b, ln_g, ln_b, edge_index, batch):
    posT = jnp.pad(pos.T, ((0, 1), (0, NPAD - N)))
    pxh = posT[0]
    pyh = posT[1]
    pzh = posT[2]
    w1a = msg_w1[:H]
    w1b = msg_w1[H:2 * H]
    wd8 = jnp.zeros((8, H), _f32).at[3].set(msg_w1[2 * H])
    w1e = msg_w1[2 * H + 1:]
    ii = edge_index[0]
    jj = edge_index[1]

    ta, tb = _make_tables(h, w1a, w1b, msg_b1.reshape(1, H))
    g, rel8 = _sc_gather(ta, tb, pxh, pyh, pzh, ii, jj)
    m, cu8 = _edge_mlp(g, rel8, edge_attr, wd8, w1e, msg_w2,
                       msg_b2.reshape(1, H), coord_w1, coord_b1.reshape(1, H),
                       coord_w2.reshape(1, H))
    parts, cpx, cpy, cpz = _sc_scatter(m, cu8, ii)
    h_new, posnT = _node_mlp(
        h, parts, cpx, cpy, cpz, posT, batch.reshape(N, 1), cond, film_w,
        film_b.reshape(1, 2 * H), node_w1[:H], node_w1[H:],
        node_b1.reshape(1, H), node_w2, node_b2.reshape(1, H),
        ln_g.reshape(1, H), ln_b.reshape(1, H))
    return h_new, posnT[:3, :N].T


# speedup vs baseline: 6.6326x; 1.0176x over previous
import functools

import jax
import jax.numpy as jnp
from jax import lax
from jax.experimental import pallas as pl
from jax.experimental.pallas import tpu as pltpu
from jax.experimental.pallas import tpu_sc as plsc

N = 10000
E = 320000
H = 128
ED = 16
G = 64
CE = 128
NW = 32
FULL = (E // NW) // CE
EPW = FULL * CE
TAIL_BASE = NW * EPW
TAIL_CHUNKS = (E - TAIL_BASE) // CE
NG = CE // 16

BE = 2560
BN = 1000
NPAD = 10240
RPT = NPAD // 16
ZR = 128

_f32 = jnp.float32


def _silu(x):
    return x * jax.lax.logistic(x)


def _tables_body(h_ref, w1a_ref, w1b_ref, b1_ref, ta_ref, tb_ref):
    hb = h_ref[...]
    ta_ref[...] = jnp.dot(hb, w1a_ref[...], preferred_element_type=_f32) + b1_ref[...]
    tb_ref[...] = jnp.dot(hb, w1b_ref[...], preferred_element_type=_f32)


def _make_tables(h, w1a, w1b, b1):
    return pl.pallas_call(
        _tables_body,
        grid=(N // BN,),
        in_specs=[
            pl.BlockSpec((BN, H), lambda i: (i, 0)),
            pl.BlockSpec((H, H), lambda i: (0, 0)),
            pl.BlockSpec((H, H), lambda i: (0, 0)),
            pl.BlockSpec((1, H), lambda i: (0, 0)),
        ],
        out_specs=[
            pl.BlockSpec((BN, H), lambda i: (i, 0)),
            pl.BlockSpec((BN, H), lambda i: (i, 0)),
        ],
        out_shape=[
            jax.ShapeDtypeStruct((N, H), _f32),
            jax.ShapeDtypeStruct((N, H), _f32),
        ],
    )(h, w1a, w1b, b1)


def _gather_body(ta, tb, pxh, pyh, pzh, eidx, g, rel8, *scr):
    c = lax.axis_index("c")
    s = lax.axis_index("s")
    wid = s * 2 + c
    S0 = scr[:12]
    S1 = scr[12:24]
    nch = FULL + jnp.where(wid < TAIL_CHUNKS, 1, 0)

    def base_of(t):
        return pl.multiple_of(
            jnp.where(t < FULL, wid * EPW + t * CE, TAIL_BASE + wid * CE), CE)

    for S in (S0, S1):
        bufr = S[3]
        for d in range(4, 8):
            for gg in range(NG):
                bufr[d, pl.ds(gg * 16, 16)] = jnp.zeros((16,), _f32)

    def gather_copies(S):
        idx2, bufa, bufb, bufr, pxi, pyi, pzi, pxj, pyj, pzj, sg = S[:11]
        idxi = idx2.at[0]
        idxj = idx2.at[1]
        return [pltpu.make_async_copy(ta.at[idxi], bufa, sg),
                pltpu.make_async_copy(tb.at[idxj], bufb, sg),
                pltpu.make_async_copy(pxh.at[idxi], pxi, sg),
                pltpu.make_async_copy(pyh.at[idxi], pyi, sg),
                pltpu.make_async_copy(pzh.at[idxi], pzi, sg),
                pltpu.make_async_copy(pxh.at[idxj], pxj, sg),
                pltpu.make_async_copy(pyh.at[idxj], pyj, sg),
                pltpu.make_async_copy(pzh.at[idxj], pzj, sg)]

    def idx_copies(S, base):
        return [pltpu.make_async_copy(eidx.at[:, pl.ds(base, CE)], S[0], S[11])]

    def start_idx(S, base):
        for cp in idx_copies(S, base):
            cp.start()

    def start_g(S, base, sw):
        for cp in idx_copies(S, base):
            cp.wait()
        for cp in gather_copies(S):
            cp.start()

    def wait_g(S):
        for cp in gather_copies(S):
            cp.wait()

    def write_copies(S, base, sw):
        bufa, bufr = S[1], S[3]
        return [pltpu.make_async_copy(bufa, g.at[pl.ds(base, CE)], sw),
                pltpu.make_async_copy(bufr, rel8.at[:, pl.ds(base, CE)], sw)]

    def compute(S):
        idx2, bufa, bufb, bufr, pxi, pyi, pzi, pxj, pyj, pzj, sg = S[:11]

        def esum(e, carry):
            for k in range(H // 16):
                dk = pl.ds(k * 16, 16)
                bufa[e, dk] = bufa[e, dk] + bufb[e, dk]
            return carry

        lax.fori_loop(0, CE, esum, 0)
        for gg in range(NG):
            d16 = pl.ds(gg * 16, 16)
            rx = pxi[d16] - pxj[d16]
            ry = pyi[d16] - pyj[d16]
            rz = pzi[d16] - pzj[d16]
            bufr[0, d16] = rx
            bufr[1, d16] = ry
            bufr[2, d16] = rz
            bufr[3, d16] = rx * rx + ry * ry + rz * rz

    sw0, sw1 = scr[24], scr[25]
    start_idx(S0, base_of(0))
    start_g(S0, base_of(0), sw0)
    start_idx(S1, base_of(1))

    def pair(p, carry):
        t0 = 2 * p
        wait_g(S0)

        @pl.when(p > 0)
        def _():
            for cp in write_copies(S1, base_of(t0 - 1), sw1):
                cp.wait()

        start_g(S1, base_of(t0 + 1), sw1)

        @pl.when(t0 + 2 < nch)
        def _():
            start_idx(S0, base_of(t0 + 2))

        compute(S0)
        for cp in write_copies(S0, base_of(t0), sw0):
            cp.start()
        wait_g(S1)
        for cp in write_copies(S0, base_of(t0), sw0):
            cp.wait()

        @pl.when(t0 + 2 < nch)
        def _():
            start_g(S0, base_of(t0 + 2), sw0)

        @pl.when(t0 + 3 < nch)
        def _():
            start_idx(S1, base_of(t0 + 3))

        compute(S1)
        for cp in write_copies(S1, base_of(t0 + 1), sw1):
            cp.start()
        return carry

    lax.fori_loop(0, FULL // 2, pair, 0)

    @pl.when(nch > FULL)
    def _():
        wait_g(S0)
        for cp in write_copies(S1, base_of(FULL - 1), sw1):
            cp.wait()
        compute(S0)
        for cp in write_copies(S0, base_of(FULL), sw0):
            cp.start()
        for cp in write_copies(S0, base_of(FULL), sw0):
            cp.wait()

    @pl.when(nch == FULL)
    def _():
        for cp in write_copies(S1, base_of(FULL - 1), sw1):
            cp.wait()


def _sc_gather(ta, tb, pxh, pyh, pzh, eidx):
    mesh = plsc.VectorSubcoreMesh(core_axis_name="c", subcore_axis_name="s")
    bufset = [
        pltpu.VMEM((2, CE), jnp.int32),
        pltpu.VMEM((CE, H), _f32),
        pltpu.VMEM((CE, H), _f32),
        pltpu.VMEM((8, CE), _f32),
        pltpu.VMEM((CE,), _f32),
        pltpu.VMEM((CE,), _f32),
        pltpu.VMEM((CE,), _f32),
        pltpu.VMEM((CE,), _f32),
        pltpu.VMEM((CE,), _f32),
        pltpu.VMEM((CE,), _f32),
        pltpu.SemaphoreType.DMA,
        pltpu.SemaphoreType.DMA,
    ]
    f = pl.kernel(
        _gather_body,
        mesh=mesh,
        out_type=[
            jax.ShapeDtypeStruct((E, H), _f32),
            jax.ShapeDtypeStruct((8, E), _f32),
        ],
        scratch_types=bufset + bufset + [pltpu.SemaphoreType.DMA,
                                         pltpu.SemaphoreType.DMA],
    )
    return f(ta, tb, pxh, pyh, pzh, eidx)


def _edge_body(g_ref, r8_ref, ea_ref, wd8_ref, w1e_ref, w2_ref,
               b2_ref, cw1_ref, cb1_ref, cw2_ref, m_ref, cu_ref):
    r8 = r8_ref[...]
    pre = (g_ref[...]
           + jnp.dot(ea_ref[...], w1e_ref[...], preferred_element_type=_f32)
           + lax.dot_general(r8, wd8_ref[...], (((0,), (0,)), ((), ())),
                             preferred_element_type=_f32))
    m1 = _silu(pre)
    m2 = _silu(jnp.dot(m1, w2_ref[...], preferred_element_type=_f32) + b2_ref[...])
    t = _silu(jnp.dot(m2, cw1_ref[...], preferred_element_type=_f32) + cb1_ref[...])
    cw_row = lax.dot_general(cw2_ref[...], t, (((1,), (1,)), ((), ())),
                             preferred_element_type=_f32)
    dsq_row = r8[3:4, :]
    scale = cw_row * lax.rsqrt(dsq_row + 1e-8)
    cu = r8[0:3, :] * scale
    m_ref[...] = m2
    cu_ref[...] = jnp.concatenate(
        [cu, jnp.zeros((5, cu.shape[1]), _f32)], axis=0)


def _edge_mlp(g, rel8, ea, wd8, w1e, w2, b2, cw1, cb1, cw2):
    full = lambda i: (0, 0)
    return pl.pallas_call(
        _edge_body,
        grid=(E // BE,),
        in_specs=[
            pl.BlockSpec((BE, H), lambda i: (i, 0)),
            pl.BlockSpec((8, BE), lambda i: (0, i)),
            pl.BlockSpec((BE, ED), lambda i: (i, 0)),
            pl.BlockSpec((8, H), full),
            pl.BlockSpec((ED, H), full),
            pl.BlockSpec((H, H), full),
            pl.BlockSpec((1, H), full),
            pl.BlockSpec((H, H), full),
            pl.BlockSpec((1, H), full),
            pl.BlockSpec((1, H), full),
        ],
        out_specs=[
            pl.BlockSpec((BE, H), lambda i: (i, 0)),
            pl.BlockSpec((8, BE), lambda i: (0, i)),
        ],
        out_shape=[
            jax.ShapeDtypeStruct((E, H), _f32),
            jax.ShapeDtypeStruct((8, E), _f32),
        ],
    )(g, rel8, ea, wd8, w1e, w2, b2, cw1, cb1, cw2)


def _scatter_body(m, cu8, ii, parts, cpx, cpy, cpz,
                  idxv, mbuf, cbuf, sl0, idxv1, mbuf1, cbuf1, sl1,
                  zbuf1, acc, accx, accy, accz):
    c = lax.axis_index("c")
    s = lax.axis_index("s")
    wid = s * 2 + c

    def zb_body(r, carry):
        for k in range(H // 16):
            mbuf[r, pl.ds(k * 16, 16)] = jnp.zeros((16,), _f32)
        return carry

    lax.fori_loop(0, CE, zb_body, 0)

    def z1_body(r, carry):
        zbuf1[pl.ds(r * 16, 16)] = jnp.zeros((16,), _f32)
        return carry

    lax.fori_loop(0, RPT // 16, z1_body, 0)
    for q in range(RPT // CE):
        pltpu.sync_copy(mbuf, acc.at[pl.ds(s * RPT + q * CE, CE)])
    pltpu.sync_copy(zbuf1, accx.at[pl.ds(s * RPT, RPT)])
    pltpu.sync_copy(zbuf1, accy.at[pl.ds(s * RPT, RPT)])
    pltpu.sync_copy(zbuf1, accz.at[pl.ds(s * RPT, RPT)])
    plsc.subcore_barrier()

    nch = FULL + jnp.where(wid < TAIL_CHUNKS, 1, 0)

    def base_of(t):
        return pl.multiple_of(
            jnp.where(t < FULL, wid * EPW + t * CE, TAIL_BASE + wid * CE), CE)

    def load_copies(S, base):
        idxv, mbuf, cbuf, sl = S
        return [pltpu.make_async_copy(ii.at[pl.ds(base, CE)], idxv, sl),
                pltpu.make_async_copy(m.at[pl.ds(base, CE)], mbuf, sl),
                pltpu.make_async_copy(cu8.at[:, pl.ds(base, CE)], cbuf, sl)]

    def scatter4(S):
        idxv, mbuf, cbuf, sl = S
        pltpu.sync_copy(mbuf, acc.at[idxv], add=True)
        pltpu.sync_copy(cbuf.at[0], accx.at[idxv], add=True)
        pltpu.sync_copy(cbuf.at[1], accy.at[idxv], add=True)
        pltpu.sync_copy(cbuf.at[2], accz.at[idxv], add=True)

    S0 = (idxv, mbuf, cbuf, sl0)
    S1 = (idxv1, mbuf1, cbuf1, sl1)
    for cp in load_copies(S0, base_of(0)):
        cp.start()

    def pair(p, carry):
        t0 = 2 * p
        for cp in load_copies(S0, base_of(t0)):
            cp.wait()
        for cp in load_copies(S1, base_of(t0 + 1)):
            cp.start()
        scatter4(S0)
        for cp in load_copies(S1, base_of(t0 + 1)):
            cp.wait()

        @pl.when(t0 + 2 < nch)
        def _():
            for cp in load_copies(S0, base_of(t0 + 2)):
                cp.start()

        scatter4(S1)
        return carry

    lax.fori_loop(0, FULL // 2, pair, 0)

    @pl.when(nch > FULL)
    def _():
        for cp in load_copies(S0, base_of(FULL)):
            cp.wait()
        scatter4(S0)

    plsc.subcore_barrier()
    pltpu.sync_copy(acc.at[pl.ds(s * RPT, RPT)],
                    parts.at[c, pl.ds(s * RPT, RPT)])

    @pl.when(s == 0)
    def _():
        pltpu.sync_copy(accx, cpx.at[c])
        pltpu.sync_copy(accy, cpy.at[c])
        pltpu.sync_copy(accz, cpz.at[c])


def _sc_scatter(m, cu8, ii):
    mesh = plsc.VectorSubcoreMesh(core_axis_name="c", subcore_axis_name="s")
    f = pl.kernel(
        _scatter_body,
        mesh=mesh,
        out_type=[
            jax.ShapeDtypeStruct((2, NPAD, H), _f32),
            jax.ShapeDtypeStruct((2, NPAD), _f32),
            jax.ShapeDtypeStruct((2, NPAD), _f32),
            jax.ShapeDtypeStruct((2, NPAD), _f32),
        ],
        scratch_types=[
            pltpu.VMEM((CE,), jnp.int32),
            pltpu.VMEM((CE, H), _f32),
            pltpu.VMEM((8, CE), _f32),
            pltpu.SemaphoreType.DMA,
            pltpu.VMEM((CE,), jnp.int32),
            pltpu.VMEM((CE, H), _f32),
            pltpu.VMEM((8, CE), _f32),
            pltpu.SemaphoreType.DMA,
            pltpu.VMEM((RPT,), _f32),
            pltpu.MemorySpace.VMEM_SHARED((NPAD, H), _f32),
            pltpu.MemorySpace.VMEM_SHARED((NPAD,), _f32),
            pltpu.MemorySpace.VMEM_SHARED((NPAD,), _f32),
            pltpu.MemorySpace.VMEM_SHARED((NPAD,), _f32),
        ],
    )
    return f(m, cu8, ii)


def _node_body(h_ref, parts_ref, cpx_ref, cpy_ref, cpz_ref, pos3_ref,
               batch_ref, cond_ref,
               fw_ref, fb_ref, nw1a_ref, nw1b_ref, nb1_ref, nw2_ref, nb2_ref,
               lng_ref, lnb_ref, hnew_ref, posn_ref):
    agg = parts_ref[0] + parts_ref[1]
    hb = h_ref[...]
    hn = _silu(jnp.dot(hb, nw1a_ref[...], preferred_element_type=_f32)
               + jnp.dot(agg, nw1b_ref[...], preferred_element_type=_f32)
               + nb1_ref[...])
    h2 = jnp.dot(hn, nw2_ref[...], preferred_element_type=_f32) + nb2_ref[...]
    film = jnp.dot(cond_ref[...], fw_ref[...], preferred_element_type=_f32) + fb_ref[...]
    oh = (batch_ref[...] == lax.broadcasted_iota(jnp.int32, (BN, G), 1)).astype(_f32)
    gbt = jnp.dot(oh, film, preferred_element_type=_f32)
    h2 = gbt[:, :H] * h2 + gbt[:, H:]
    r = hb + h2
    mu = jnp.mean(r, axis=1, keepdims=True)
    var = jnp.mean((r - mu) * (r - mu), axis=1, keepdims=True)
    hnew_ref[...] = (r - mu) * lax.rsqrt(var + 1e-5) * lng_ref[...] + lnb_ref[...]
    cu = jnp.concatenate(
        [cpx_ref[0:1, :] + cpx_ref[1:2, :],
         cpy_ref[0:1, :] + cpy_ref[1:2, :],
         cpz_ref[0:1, :] + cpz_ref[1:2, :],
         jnp.zeros((1, NPAD), _f32)], axis=0)
    posn_ref[...] = pos3_ref[...] + cu


def _node_mlp(h, parts, cpx, cpy, cpz, pos3T, batch2, cond, fw, fb,
              nw1a, nw1b, nb1, nw2, nb2, lng, lnb):
    full = lambda i: (0, 0)
    return pl.pallas_call(
        _node_body,
        grid=(N // BN,),
        in_specs=[
            pl.BlockSpec((BN, H), lambda i: (i, 0)),
            pl.BlockSpec((2, BN, H), lambda i: (0, i, 0)),
            pl.BlockSpec((2, NPAD), full),
            pl.BlockSpec((2, NPAD), full),
            pl.BlockSpec((2, NPAD), full),
            pl.BlockSpec((4, NPAD), full),
            pl.BlockSpec((BN, 1), lambda i: (i, 0)),
            pl.BlockSpec((G, 128), full),
            pl.BlockSpec((128, 2 * H), full),
            pl.BlockSpec((1, 2 * H), full),
            pl.BlockSpec((H, H), full),
            pl.BlockSpec((H, H), full),
            pl.BlockSpec((1, H), full),
            pl.BlockSpec((H, H), full),
            pl.BlockSpec((1, H), full),
            pl.BlockSpec((1, H), full),
            pl.BlockSpec((1, H), full),
        ],
        out_specs=[
            pl.BlockSpec((BN, H), lambda i: (i, 0)),
            pl.BlockSpec((4, NPAD), lambda i: (0, 0)),
        ],
        out_shape=[
            jax.ShapeDtypeStruct((N, H), _f32),
            jax.ShapeDtypeStruct((4, NPAD), _f32),
        ],
    )(h, parts, cpx, cpy, cpz, pos3T, batch2, cond, fw, fb,
      nw1a, nw1b, nb1, nw2, nb2, lng, lnb)


def kernel(h, pos, edge_attr, cond, msg_w1, msg_b1, msg_w2, msg_b2,
           coord_w1, coord_b1, coord_w2, node_w1, node_b1, node_w2, node_b2,
           film_w, film_b, ln_g, ln_b, edge_index, batch):
    posT = jnp.pad(pos.T, ((0, 1), (0, NPAD - N)))
    pxh = posT[0]
    pyh = posT[1]
    pzh = posT[2]
    w1a = msg_w1[:H]
    w1b = msg_w1[H:2 * H]
    wd8 = jnp.zeros((8, H), _f32).at[3].set(msg_w1[2 * H])
    w1e = msg_w1[2 * H + 1:]
    ii = edge_index[0]
    jj = edge_index[1]

    ta, tb = _make_tables(h, w1a, w1b, msg_b1.reshape(1, H))
    g, rel8 = _sc_gather(ta, tb, pxh, pyh, pzh, edge_index)
    m, cu8 = _edge_mlp(g, rel8, edge_attr, wd8, w1e, msg_w2,
                       msg_b2.reshape(1, H), coord_w1, coord_b1.reshape(1, H),
                       coord_w2.reshape(1, H))
    parts, cpx, cpy, cpz = _sc_scatter(m, cu8, ii)
    h_new, posnT = _node_mlp(
        h, parts, cpx, cpy, cpz, posT, batch.reshape(N, 1), cond, film_w,
        film_b.reshape(1, 2 * H), node_w1[:H], node_w1[H:],
        node_b1.reshape(1, H), node_w2, node_b2.reshape(1, H),
        ln_g.reshape(1, H), ln_b.reshape(1, H))
    return h_new, posnT[:3, :N].T


# speedup vs baseline: 7.1841x; 1.0831x over previous
import functools

import jax
import jax.numpy as jnp
from jax import lax
from jax.experimental import pallas as pl
from jax.experimental.pallas import tpu as pltpu
from jax.experimental.pallas import tpu_sc as plsc

N = 10000
E = 320000
H = 128
ED = 16
G = 64
CE = 128
NW = 32
NG = CE // 16
SEG1 = 163840
SEGS = ((0, SEG1, 40, 0), (SEG1, E - SEG1, 38, 4))

BE = 2560
BN = 1000
NPAD = 10240
RPT = NPAD // 16
ZR = 128

_f32 = jnp.float32


def _silu(x):
    return x * jax.lax.logistic(x)


def _tables_body(h_ref, w1a_ref, w1b_ref, b1_ref, ta_ref, tb_ref):
    hb = h_ref[...]
    ta_ref[...] = jnp.dot(hb, w1a_ref[...], preferred_element_type=_f32) + b1_ref[...]
    tb_ref[...] = jnp.dot(hb, w1b_ref[...], preferred_element_type=_f32)


def _make_tables(h, w1a, w1b, b1):
    return pl.pallas_call(
        _tables_body,
        grid=(N // BN,),
        in_specs=[
            pl.BlockSpec((BN, H), lambda i: (i, 0)),
            pl.BlockSpec((H, H), lambda i: (0, 0)),
            pl.BlockSpec((H, H), lambda i: (0, 0)),
            pl.BlockSpec((1, H), lambda i: (0, 0)),
        ],
        out_specs=[
            pl.BlockSpec((BN, H), lambda i: (i, 0)),
            pl.BlockSpec((BN, H), lambda i: (i, 0)),
        ],
        out_shape=[
            jax.ShapeDtypeStruct((N, H), _f32),
            jax.ShapeDtypeStruct((N, H), _f32),
        ],
    )(h, w1a, w1b, b1)


def _gather_body(start, full, tailc, ta, tb, pxh, pyh, pzh, eidx, g, rel8,
                 *scr):
    c = lax.axis_index("c")
    s = lax.axis_index("s")
    wid = s * 2 + c
    epw = full * CE
    S0 = scr[:12]
    S1 = scr[12:24]
    nch = full + jnp.where(wid < tailc, 1, 0)

    def rel_of(t):
        return pl.multiple_of(
            jnp.where(t < full, wid * epw + t * CE, NW * epw + wid * CE), CE)

    def base_of(t):
        return pl.multiple_of(start + rel_of(t), CE)

    for S in (S0, S1):
        bufr = S[3]
        for d in range(4, 8):
            for gg in range(NG):
                bufr[d, pl.ds(gg * 16, 16)] = jnp.zeros((16,), _f32)

    def gather_copies(S):
        idx2, bufa, bufb, bufr, pxi, pyi, pzi, pxj, pyj, pzj, sg = S[:11]
        idxi = idx2.at[0]
        idxj = idx2.at[1]
        return [pltpu.make_async_copy(ta.at[idxi], bufa, sg),
                pltpu.make_async_copy(tb.at[idxj], bufb, sg),
                pltpu.make_async_copy(pxh.at[idxi], pxi, sg),
                pltpu.make_async_copy(pyh.at[idxi], pyi, sg),
                pltpu.make_async_copy(pzh.at[idxi], pzi, sg),
                pltpu.make_async_copy(pxh.at[idxj], pxj, sg),
                pltpu.make_async_copy(pyh.at[idxj], pyj, sg),
                pltpu.make_async_copy(pzh.at[idxj], pzj, sg)]

    def idx_copies(S, t):
        base = base_of(t)
        return [pltpu.make_async_copy(eidx.at[:, pl.ds(base, CE)], S[0], S[11])]

    def start_idx(S, t):
        for cp in idx_copies(S, t):
            cp.start()

    def start_g(S, t, sw):
        for cp in idx_copies(S, t):
            cp.wait()
        for cp in gather_copies(S):
            cp.start()

    def wait_g(S):
        for cp in gather_copies(S):
            cp.wait()

    def write_copies(S, t, sw):
        base = rel_of(t)
        bufa, bufr = S[1], S[3]
        return [pltpu.make_async_copy(bufa, g.at[pl.ds(base, CE)], sw),
                pltpu.make_async_copy(bufr, rel8.at[:, pl.ds(base, CE)], sw)]

    def compute(S):
        idx2, bufa, bufb, bufr, pxi, pyi, pzi, pxj, pyj, pzj, sg = S[:11]

        def esum(e, carry):
            for k in range(H // 16):
                dk = pl.ds(k * 16, 16)
                bufa[e, dk] = bufa[e, dk] + bufb[e, dk]
            return carry

        lax.fori_loop(0, CE, esum, 0)
        for gg in range(NG):
            d16 = pl.ds(gg * 16, 16)
            rx = pxi[d16] - pxj[d16]
            ry = pyi[d16] - pyj[d16]
            rz = pzi[d16] - pzj[d16]
            bufr[0, d16] = rx
            bufr[1, d16] = ry
            bufr[2, d16] = rz
            bufr[3, d16] = rx * rx + ry * ry + rz * rz

    sw0, sw1 = scr[24], scr[25]
    start_idx(S0, 0)
    start_g(S0, 0, sw0)
    start_idx(S1, 1)

    def pair(p, carry):
        t0 = 2 * p
        wait_g(S0)

        @pl.when(p > 0)
        def _():
            for cp in write_copies(S1, t0 - 1, sw1):
                cp.wait()

        start_g(S1, t0 + 1, sw1)

        @pl.when(t0 + 2 < nch)
        def _():
            start_idx(S0, t0 + 2)

        compute(S0)
        for cp in write_copies(S0, t0, sw0):
            cp.start()
        wait_g(S1)
        for cp in write_copies(S0, t0, sw0):
            cp.wait()

        @pl.when(t0 + 2 < nch)
        def _():
            start_g(S0, t0 + 2, sw0)

        @pl.when(t0 + 3 < nch)
        def _():
            start_idx(S1, t0 + 3)

        compute(S1)
        for cp in write_copies(S1, t0 + 1, sw1):
            cp.start()
        return carry

    lax.fori_loop(0, full // 2, pair, 0)

    @pl.when(nch > full)
    def _():
        wait_g(S0)
        for cp in write_copies(S1, full - 1, sw1):
            cp.wait()
        compute(S0)
        for cp in write_copies(S0, full, sw0):
            cp.start()
        for cp in write_copies(S0, full, sw0):
            cp.wait()

    @pl.when(nch == full)
    def _():
        for cp in write_copies(S1, full - 1, sw1):
            cp.wait()


def _sc_gather(seg, ta, tb, pxh, pyh, pzh, eidx):
    start, seglen, full, tailc = seg
    mesh = plsc.VectorSubcoreMesh(core_axis_name="c", subcore_axis_name="s")
    bufset = [
        pltpu.VMEM((2, CE), jnp.int32),
        pltpu.VMEM((CE, H), _f32),
        pltpu.VMEM((CE, H), _f32),
        pltpu.VMEM((8, CE), _f32),
        pltpu.VMEM((CE,), _f32),
        pltpu.VMEM((CE,), _f32),
        pltpu.VMEM((CE,), _f32),
        pltpu.VMEM((CE,), _f32),
        pltpu.VMEM((CE,), _f32),
        pltpu.VMEM((CE,), _f32),
        pltpu.SemaphoreType.DMA,
        pltpu.SemaphoreType.DMA,
    ]
    f = pl.kernel(
        functools.partial(_gather_body, start, full, tailc),
        mesh=mesh,
        out_type=[
            jax.ShapeDtypeStruct((seglen, H), _f32),
            jax.ShapeDtypeStruct((8, seglen), _f32),
        ],
        scratch_types=bufset + bufset + [pltpu.SemaphoreType.DMA,
                                         pltpu.SemaphoreType.DMA],
    )
    return f(ta, tb, pxh, pyh, pzh, eidx)


def _edge_body(g_ref, r8_ref, ea_ref, wd8_ref, w1e_ref, w2_ref,
               b2_ref, cw1_ref, cb1_ref, cw2_ref, m_ref, cu_ref):
    r8 = r8_ref[...]
    pre = (g_ref[...]
           + jnp.dot(ea_ref[...], w1e_ref[...], preferred_element_type=_f32)
           + lax.dot_general(r8, wd8_ref[...], (((0,), (0,)), ((), ())),
                             preferred_element_type=_f32))
    m1 = _silu(pre)
    m2 = _silu(jnp.dot(m1, w2_ref[...], preferred_element_type=_f32) + b2_ref[...])
    t = _silu(jnp.dot(m2, cw1_ref[...], preferred_element_type=_f32) + cb1_ref[...])
    cw_row = lax.dot_general(cw2_ref[...], t, (((1,), (1,)), ((), ())),
                             preferred_element_type=_f32)
    dsq_row = r8[3:4, :]
    scale = cw_row * lax.rsqrt(dsq_row + 1e-8)
    cu = r8[0:3, :] * scale
    m_ref[...] = m2
    cu_ref[...] = jnp.concatenate(
        [cu, jnp.zeros((5, cu.shape[1]), _f32)], axis=0)


def _edge_mlp(g, rel8, ea, wd8, w1e, w2, b2, cw1, cb1, cw2):
    full = lambda i: (0, 0)
    seglen = g.shape[0]
    return pl.pallas_call(
        _edge_body,
        grid=(seglen // BE,),
        in_specs=[
            pl.BlockSpec((BE, H), lambda i: (i, 0)),
            pl.BlockSpec((8, BE), lambda i: (0, i)),
            pl.BlockSpec((BE, ED), lambda i: (i, 0)),
            pl.BlockSpec((8, H), full),
            pl.BlockSpec((ED, H), full),
            pl.BlockSpec((H, H), full),
            pl.BlockSpec((1, H), full),
            pl.BlockSpec((H, H), full),
            pl.BlockSpec((1, H), full),
            pl.BlockSpec((1, H), full),
        ],
        out_specs=[
            pl.BlockSpec((BE, H), lambda i: (i, 0)),
            pl.BlockSpec((8, BE), lambda i: (0, i)),
        ],
        out_shape=[
            jax.ShapeDtypeStruct((seglen, H), _f32),
            jax.ShapeDtypeStruct((8, seglen), _f32),
        ],
    )(g, rel8, ea, wd8, w1e, w2, b2, cw1, cb1, cw2)


def _scatter_body(start, full, tailc, m, cu8, ii, parts, cpx, cpy, cpz,
                  idxv, mbuf, cbuf, sl0, idxv1, mbuf1, cbuf1, sl1,
                  zbuf1, acc, accx, accy, accz):
    c = lax.axis_index("c")
    s = lax.axis_index("s")
    wid = s * 2 + c

    def zb_body(r, carry):
        for k in range(H // 16):
            mbuf[r, pl.ds(k * 16, 16)] = jnp.zeros((16,), _f32)
        return carry

    lax.fori_loop(0, CE, zb_body, 0)

    def z1_body(r, carry):
        zbuf1[pl.ds(r * 16, 16)] = jnp.zeros((16,), _f32)
        return carry

    lax.fori_loop(0, RPT // 16, z1_body, 0)
    for q in range(RPT // CE):
        pltpu.sync_copy(mbuf, acc.at[pl.ds(s * RPT + q * CE, CE)])
    pltpu.sync_copy(zbuf1, accx.at[pl.ds(s * RPT, RPT)])
    pltpu.sync_copy(zbuf1, accy.at[pl.ds(s * RPT, RPT)])
    pltpu.sync_copy(zbuf1, accz.at[pl.ds(s * RPT, RPT)])
    plsc.subcore_barrier()

    epw = full * CE
    nch = full + jnp.where(wid < tailc, 1, 0)

    def rel_of(t):
        return pl.multiple_of(
            jnp.where(t < full, wid * epw + t * CE, NW * epw + wid * CE), CE)

    def load_copies(S, t):
        rbase = rel_of(t)
        abase = pl.multiple_of(start + rbase, CE)
        idxv, mbuf, cbuf, sl = S
        return [pltpu.make_async_copy(ii.at[pl.ds(abase, CE)], idxv, sl),
                pltpu.make_async_copy(m.at[pl.ds(rbase, CE)], mbuf, sl),
                pltpu.make_async_copy(cu8.at[:, pl.ds(rbase, CE)], cbuf, sl)]

    def scatter4(S):
        idxv, mbuf, cbuf, sl = S
        pltpu.sync_copy(mbuf, acc.at[idxv], add=True)
        pltpu.sync_copy(cbuf.at[0], accx.at[idxv], add=True)
        pltpu.sync_copy(cbuf.at[1], accy.at[idxv], add=True)
        pltpu.sync_copy(cbuf.at[2], accz.at[idxv], add=True)

    S0 = (idxv, mbuf, cbuf, sl0)
    S1 = (idxv1, mbuf1, cbuf1, sl1)
    for cp in load_copies(S0, 0):
        cp.start()

    def pair(p, carry):
        t0 = 2 * p
        for cp in load_copies(S0, t0):
            cp.wait()
        for cp in load_copies(S1, t0 + 1):
            cp.start()
        scatter4(S0)
        for cp in load_copies(S1, t0 + 1):
            cp.wait()

        @pl.when(t0 + 2 < nch)
        def _():
            for cp in load_copies(S0, t0 + 2):
                cp.start()

        scatter4(S1)
        return carry

    lax.fori_loop(0, full // 2, pair, 0)

    @pl.when(nch > full)
    def _():
        for cp in load_copies(S0, full):
            cp.wait()
        scatter4(S0)

    plsc.subcore_barrier()
    pltpu.sync_copy(acc.at[pl.ds(s * RPT, RPT)],
                    parts.at[c, pl.ds(s * RPT, RPT)])

    @pl.when(s == 0)
    def _():
        pltpu.sync_copy(accx, cpx.at[c])
        pltpu.sync_copy(accy, cpy.at[c])
        pltpu.sync_copy(accz, cpz.at[c])


def _sc_scatter(seg, m, cu8, ii):
    start, seglen, full, tailc = seg
    mesh = plsc.VectorSubcoreMesh(core_axis_name="c", subcore_axis_name="s")
    f = pl.kernel(
        functools.partial(_scatter_body, start, full, tailc),
        mesh=mesh,
        out_type=[
            jax.ShapeDtypeStruct((2, NPAD, H), _f32),
            jax.ShapeDtypeStruct((2, NPAD), _f32),
            jax.ShapeDtypeStruct((2, NPAD), _f32),
            jax.ShapeDtypeStruct((2, NPAD), _f32),
        ],
        scratch_types=[
            pltpu.VMEM((CE,), jnp.int32),
            pltpu.VMEM((CE, H), _f32),
            pltpu.VMEM((8, CE), _f32),
            pltpu.SemaphoreType.DMA,
            pltpu.VMEM((CE,), jnp.int32),
            pltpu.VMEM((CE, H), _f32),
            pltpu.VMEM((8, CE), _f32),
            pltpu.SemaphoreType.DMA,
            pltpu.VMEM((RPT,), _f32),
            pltpu.MemorySpace.VMEM_SHARED((NPAD, H), _f32),
            pltpu.MemorySpace.VMEM_SHARED((NPAD,), _f32),
            pltpu.MemorySpace.VMEM_SHARED((NPAD,), _f32),
            pltpu.MemorySpace.VMEM_SHARED((NPAD,), _f32),
        ],
    )
    return f(m, cu8, ii)


def _node_body(h_ref, parts_ref, parts2_ref, cpx_ref, cpy_ref, cpz_ref,
               cpx2_ref, cpy2_ref, cpz2_ref, pos3_ref,
               batch_ref, cond_ref,
               fw_ref, fb_ref, nw1a_ref, nw1b_ref, nb1_ref, nw2_ref, nb2_ref,
               lng_ref, lnb_ref, hnew_ref, posn_ref):
    agg = (parts_ref[0] + parts_ref[1]) + (parts2_ref[0] + parts2_ref[1])
    hb = h_ref[...]
    hn = _silu(jnp.dot(hb, nw1a_ref[...], preferred_element_type=_f32)
               + jnp.dot(agg, nw1b_ref[...], preferred_element_type=_f32)
               + nb1_ref[...])
    h2 = jnp.dot(hn, nw2_ref[...], preferred_element_type=_f32) + nb2_ref[...]
    film = jnp.dot(cond_ref[...], fw_ref[...], preferred_element_type=_f32) + fb_ref[...]
    oh = (batch_ref[...] == lax.broadcasted_iota(jnp.int32, (BN, G), 1)).astype(_f32)
    gbt = jnp.dot(oh, film, preferred_element_type=_f32)
    h2 = gbt[:, :H] * h2 + gbt[:, H:]
    r = hb + h2
    mu = jnp.mean(r, axis=1, keepdims=True)
    var = jnp.mean((r - mu) * (r - mu), axis=1, keepdims=True)
    hnew_ref[...] = (r - mu) * lax.rsqrt(var + 1e-5) * lng_ref[...] + lnb_ref[...]
    cu = jnp.concatenate(
        [cpx_ref[0:1, :] + cpx_ref[1:2, :] + cpx2_ref[0:1, :] + cpx2_ref[1:2, :],
         cpy_ref[0:1, :] + cpy_ref[1:2, :] + cpy2_ref[0:1, :] + cpy2_ref[1:2, :],
         cpz_ref[0:1, :] + cpz_ref[1:2, :] + cpz2_ref[0:1, :] + cpz2_ref[1:2, :],
         jnp.zeros((1, NPAD), _f32)], axis=0)
    posn_ref[...] = pos3_ref[...] + cu


def _node_mlp(h, parts, parts2, cpx, cpy, cpz, cpx2, cpy2, cpz2,
              pos3T, batch2, cond, fw, fb,
              nw1a, nw1b, nb1, nw2, nb2, lng, lnb):
    full = lambda i: (0, 0)
    return pl.pallas_call(
        _node_body,
        grid=(N // BN,),
        in_specs=[
            pl.BlockSpec((BN, H), lambda i: (i, 0)),
            pl.BlockSpec((2, BN, H), lambda i: (0, i, 0)),
            pl.BlockSpec((2, BN, H), lambda i: (0, i, 0)),
            pl.BlockSpec((2, NPAD), full),
            pl.BlockSpec((2, NPAD), full),
            pl.BlockSpec((2, NPAD), full),
            pl.BlockSpec((2, NPAD), full),
            pl.BlockSpec((2, NPAD), full),
            pl.BlockSpec((2, NPAD), full),
            pl.BlockSpec((4, NPAD), full),
            pl.BlockSpec((BN, 1), lambda i: (i, 0)),
            pl.BlockSpec((G, 128), full),
            pl.BlockSpec((128, 2 * H), full),
            pl.BlockSpec((1, 2 * H), full),
            pl.BlockSpec((H, H), full),
            pl.BlockSpec((H, H), full),
            pl.BlockSpec((1, H), full),
            pl.BlockSpec((H, H), full),
            pl.BlockSpec((1, H), full),
            pl.BlockSpec((1, H), full),
            pl.BlockSpec((1, H), full),
        ],
        out_specs=[
            pl.BlockSpec((BN, H), lambda i: (i, 0)),
            pl.BlockSpec((4, NPAD), lambda i: (0, 0)),
        ],
        out_shape=[
            jax.ShapeDtypeStruct((N, H), _f32),
            jax.ShapeDtypeStruct((4, NPAD), _f32),
        ],
    )(h, parts, parts2, cpx, cpy, cpz, cpx2, cpy2, cpz2,
      pos3T, batch2, cond, fw, fb,
      nw1a, nw1b, nb1, nw2, nb2, lng, lnb)


def kernel(h, pos, edge_attr, cond, msg_w1, msg_b1, msg_w2, msg_b2,
           coord_w1, coord_b1, coord_w2, node_w1, node_b1, node_w2, node_b2,
           film_w, film_b, ln_g, ln_b, edge_index, batch):
    posT = jnp.pad(pos.T, ((0, 1), (0, NPAD - N)))
    pxh = posT[0]
    pyh = posT[1]
    pzh = posT[2]
    w1a = msg_w1[:H]
    w1b = msg_w1[H:2 * H]
    wd8 = jnp.zeros((8, H), _f32).at[3].set(msg_w1[2 * H])
    w1e = msg_w1[2 * H + 1:]
    ii = edge_index[0]
    jj = edge_index[1]

    ta, tb = _make_tables(h, w1a, w1b, msg_b1.reshape(1, H))
    outs = []
    for seg in SEGS:
        start, seglen = seg[0], seg[1]
        g, rel8 = _sc_gather(seg, ta, tb, pxh, pyh, pzh, edge_index)
        m, cu8 = _edge_mlp(g, rel8,
                           lax.slice_in_dim(edge_attr, start, start + seglen),
                           wd8, w1e, msg_w2,
                           msg_b2.reshape(1, H), coord_w1,
                           coord_b1.reshape(1, H), coord_w2.reshape(1, H))
        outs.append(_sc_scatter(seg, m, cu8, ii))
    (parts, cpx, cpy, cpz), (parts2, cpx2, cpy2, cpz2) = outs
    h_new, posnT = _node_mlp(
        h, parts, parts2, cpx, cpy, cpz, cpx2, cpy2, cpz2,
        posT, batch.reshape(N, 1), cond, film_w,
        film_b.reshape(1, 2 * H), node_w1[:H], node_w1[H:],
        node_b1.reshape(1, H), node_w2, node_b2.reshape(1, H),
        ln_g.reshape(1, H), ln_b.reshape(1, H))
    return h_new, posnT[:3, :N].T
